# baseline (speedup 1.0000x reference)
import functools

import jax
import jax.numpy as jnp
from jax import lax
from jax.experimental import pallas as pl
from jax.experimental.pallas import tpu as pltpu
from jax.experimental.pallas import tpu_sc as plsc

N = 10000
NP = 10240
E = 320000
D = 128
NC = 2
NS = 16
NW = NC * NS
EPT = E // NW
BK = 80
NB = EPT // BK
ROWS_PER_SUB = NP // NS

_MESH = plsc.VectorSubcoreMesh(core_axis_name="c", subcore_axis_name="s")


def _segsum_body(y_hbm, src_hbm, dst_hbm, zeros_hbm, out_hbm,
                 sidx_v, didx_v, stage_v, sem, acc_sh):
    c = lax.axis_index("c")
    s = lax.axis_index("s")
    wid = s * NC + c
    pltpu.sync_copy(zeros_hbm, acc_sh.at[pl.ds(s * ROWS_PER_SUB, ROWS_PER_SUB)])
    plsc.subcore_barrier()

    def body(i, carry):
        base = wid * EPT + i * BK
        pltpu.sync_copy(src_hbm.at[pl.ds(base, BK)], sidx_v)
        pltpu.async_copy(y_hbm.at[sidx_v], stage_v, sem).wait()
        pltpu.sync_copy(dst_hbm.at[pl.ds(base, BK)], didx_v)
        pltpu.sync_copy(stage_v, acc_sh.at[didx_v], add=True)
        return carry

    lax.fori_loop(0, NB, body, 0)
    plsc.subcore_barrier()
    pltpu.sync_copy(
        acc_sh.at[pl.ds(s * ROWS_PER_SUB, ROWS_PER_SUB)],
        out_hbm.at[c, pl.ds(s * ROWS_PER_SUB, ROWS_PER_SUB)],
    )


_segsum_kernel = functools.partial(
    pl.kernel,
    out_type=jax.ShapeDtypeStruct((NC, NP, D), jnp.float32),
    mesh=_MESH,
    scratch_types=[
        pltpu.VMEM((BK,), jnp.int32),
        pltpu.VMEM((BK,), jnp.int32),
        pltpu.VMEM((BK, D), jnp.float32),
        pltpu.SemaphoreType.DMA,
        pltpu.VMEM_SHARED((NP, D), jnp.float32),
    ],
)(_segsum_body)


_RB = 1000
_GRID = (N // _RB,)


def _dinv_block(degp_ref):
    deg = 1.0 + degp_ref[0, :, 0] + degp_ref[1, :, 0]
    return lax.rsqrt(deg)[:, None]


def _tc1_body(x_ref, w_ref, degp_ref, y_ref):
    dinv = _dinv_block(degp_ref)
    y_ref[...] = jnp.dot(x_ref[...], w_ref[...],
                         preferred_element_type=jnp.float32) * dinv


def _tc2_body(s_ref, y1_ref, degp_ref, w_ref, b_ref, y2_ref):
    dinv = _dinv_block(degp_ref)
    h = jnp.maximum((s_ref[0] + s_ref[1] + y1_ref[...]) * dinv + b_ref[...], 0.0)
    y2_ref[...] = jnp.dot(h, w_ref[...],
                          preferred_element_type=jnp.float32) * dinv


def _tc3_body(s_ref, y2_ref, degp_ref, b_ref, o_ref):
    dinv = _dinv_block(degp_ref)
    o_ref[...] = (s_ref[0] + s_ref[1] + y2_ref[...]) * dinv + b_ref[...]


_row_spec = pl.BlockSpec((_RB, D), lambda i: (i, 0))
_w_spec = pl.BlockSpec((D, D), lambda i: (0, 0))
_part_spec = pl.BlockSpec((NC, _RB, D), lambda i: (0, i, 0))
_degp_spec = _part_spec
_b_spec = pl.BlockSpec((1, D), lambda i: (0, 0))

_tc1 = pl.pallas_call(
    _tc1_body,
    grid=_GRID,
    in_specs=[_row_spec, _w_spec, _degp_spec],
    out_specs=_row_spec,
    out_shape=jax.ShapeDtypeStruct((N, D), jnp.float32),
)

_tc2 = pl.pallas_call(
    _tc2_body,
    grid=_GRID,
    in_specs=[_part_spec, _row_spec, _degp_spec, _w_spec, _b_spec],
    out_specs=_row_spec,
    out_shape=jax.ShapeDtypeStruct((N, D), jnp.float32),
)

_tc3 = pl.pallas_call(
    _tc3_body,
    grid=_GRID,
    in_specs=[_part_spec, _row_spec, _degp_spec, _b_spec],
    out_specs=_row_spec,
    out_shape=jax.ShapeDtypeStruct((N, D), jnp.float32),
)


def kernel(x, edge_index, W_shared, b_shared, W_mu, b_mu, W_logvar, b_logvar):
    src = edge_index[0].astype(jnp.int32)
    dst = edge_index[1].astype(jnp.int32)
    zeros_rows = jnp.zeros((ROWS_PER_SUB, D), jnp.float32)
    ones_tbl = jnp.ones((8, D), jnp.float32)
    zsrc = jnp.zeros((E,), jnp.int32)

    degp = _segsum_kernel(ones_tbl, zsrc, dst, zeros_rows)
    y1 = _tc1(x, W_shared, degp)
    s1p = _segsum_kernel(y1, src, dst, zeros_rows)
    w_cat = jnp.concatenate([W_mu, W_logvar], axis=1)
    b_cat = jnp.concatenate([b_mu, b_logvar])[None, :]
    y2 = _tc2(s1p, y1, degp, w_cat, b_shared[None, :])
    s2p = _segsum_kernel(y2, src, dst, zeros_rows)
    out = _tc3(s2p, y2, degp, b_cat)
    return out[:, :64], out[:, 64:]


# speedup vs baseline: 15.4206x; 15.4206x over previous
import functools

import jax
import jax.numpy as jnp
from jax import lax
from jax.experimental import pallas as pl
from jax.experimental.pallas import tpu as pltpu
from jax.experimental.pallas import tpu_sc as plsc

N = 10000
NP = 10240
E = 320000
D = 128
NC = 2
NS = 16
NW = NC * NS
EPT = E // NW
BK = 80
NB = EPT // BK
ROWS_PER_SUB = NP // NS

_MESH = plsc.VectorSubcoreMesh(core_axis_name="c", subcore_axis_name="s")


def _deg_body(dst_hbm, ones_hbm, zeros_hbm, out_hbm, didx_v, stage_v, acc_sh):
    c = lax.axis_index("c")
    s = lax.axis_index("s")
    wid = s * NC + c
    pltpu.sync_copy(ones_hbm, stage_v)
    pltpu.sync_copy(zeros_hbm, acc_sh.at[pl.ds(s * ROWS_PER_SUB, ROWS_PER_SUB)])
    plsc.subcore_barrier()

    def body(i, carry):
        base = wid * EPT + i * BK
        pltpu.sync_copy(dst_hbm.at[pl.ds(base, BK)], didx_v)
        pltpu.sync_copy(stage_v, acc_sh.at[didx_v], add=True)
        return carry

    lax.fori_loop(0, NB, body, 0)
    plsc.subcore_barrier()
    pltpu.sync_copy(
        acc_sh.at[pl.ds(s * ROWS_PER_SUB, ROWS_PER_SUB)],
        out_hbm.at[c, pl.ds(s * ROWS_PER_SUB, ROWS_PER_SUB)],
    )


_deg_kernel = functools.partial(
    pl.kernel,
    out_type=jax.ShapeDtypeStruct((NC, NP, D), jnp.float32),
    mesh=_MESH,
    scratch_types=[
        pltpu.VMEM((BK,), jnp.int32),
        pltpu.VMEM((BK, D), jnp.float32),
        pltpu.VMEM_SHARED((NP, D), jnp.float32),
    ],
)(_deg_body)


def _segsum_body(y_hbm, src_hbm, dst_hbm, zeros_hbm, out_hbm,
                 sidx_v, didx_v, stage_v, sem, acc_sh):
    c = lax.axis_index("c")
    s = lax.axis_index("s")
    wid = s * NC + c
    pltpu.sync_copy(zeros_hbm, acc_sh.at[pl.ds(s * ROWS_PER_SUB, ROWS_PER_SUB)])
    plsc.subcore_barrier()

    def body(i, carry):
        base = wid * EPT + i * BK
        pltpu.sync_copy(src_hbm.at[pl.ds(base, BK)], sidx_v)
        pltpu.async_copy(y_hbm.at[sidx_v], stage_v, sem).wait()
        pltpu.sync_copy(dst_hbm.at[pl.ds(base, BK)], didx_v)
        pltpu.sync_copy(stage_v, acc_sh.at[didx_v], add=True)
        return carry

    lax.fori_loop(0, NB, body, 0)
    plsc.subcore_barrier()
    pltpu.sync_copy(
        acc_sh.at[pl.ds(s * ROWS_PER_SUB, ROWS_PER_SUB)],
        out_hbm.at[c, pl.ds(s * ROWS_PER_SUB, ROWS_PER_SUB)],
    )


_segsum_kernel = functools.partial(
    pl.kernel,
    out_type=jax.ShapeDtypeStruct((NC, NP, D), jnp.float32),
    mesh=_MESH,
    scratch_types=[
        pltpu.VMEM((BK,), jnp.int32),
        pltpu.VMEM((BK,), jnp.int32),
        pltpu.VMEM((BK, D), jnp.float32),
        pltpu.SemaphoreType.DMA,
        pltpu.VMEM_SHARED((NP, D), jnp.float32),
    ],
)(_segsum_body)


_RB = 1000
_GRID = (N // _RB,)


def _dinv_block(degp_ref):
    deg = 1.0 + degp_ref[0, :, 0] + degp_ref[1, :, 0]
    return lax.rsqrt(deg)[:, None]


def _tc1_body(x_ref, w_ref, degp_ref, y_ref):
    dinv = _dinv_block(degp_ref)
    y_ref[...] = jnp.dot(x_ref[...], w_ref[...],
                         preferred_element_type=jnp.float32) * dinv


def _tc2_body(s_ref, y1_ref, degp_ref, w_ref, b_ref, y2_ref):
    dinv = _dinv_block(degp_ref)
    h = jnp.maximum((s_ref[0] + s_ref[1] + y1_ref[...]) * dinv + b_ref[...], 0.0)
    y2_ref[...] = jnp.dot(h, w_ref[...],
                          preferred_element_type=jnp.float32) * dinv


def _tc3_body(s_ref, y2_ref, degp_ref, b_ref, o_ref):
    dinv = _dinv_block(degp_ref)
    o_ref[...] = (s_ref[0] + s_ref[1] + y2_ref[...]) * dinv + b_ref[...]


_row_spec = pl.BlockSpec((_RB, D), lambda i: (i, 0))
_w_spec = pl.BlockSpec((D, D), lambda i: (0, 0))
_part_spec = pl.BlockSpec((NC, _RB, D), lambda i: (0, i, 0))
_degp_spec = _part_spec
_b_spec = pl.BlockSpec((1, D), lambda i: (0, 0))

_tc1 = pl.pallas_call(
    _tc1_body,
    grid=_GRID,
    in_specs=[_row_spec, _w_spec, _degp_spec],
    out_specs=_row_spec,
    out_shape=jax.ShapeDtypeStruct((N, D), jnp.float32),
)

_tc2 = pl.pallas_call(
    _tc2_body,
    grid=_GRID,
    in_specs=[_part_spec, _row_spec, _degp_spec, _w_spec, _b_spec],
    out_specs=_row_spec,
    out_shape=jax.ShapeDtypeStruct((N, D), jnp.float32),
)

_tc3 = pl.pallas_call(
    _tc3_body,
    grid=_GRID,
    in_specs=[_part_spec, _row_spec, _degp_spec, _b_spec],
    out_specs=_row_spec,
    out_shape=jax.ShapeDtypeStruct((N, D), jnp.float32),
)


def kernel(x, edge_index, W_shared, b_shared, W_mu, b_mu, W_logvar, b_logvar):
    src = edge_index[0].astype(jnp.int32)
    dst = edge_index[1].astype(jnp.int32)
    zeros_rows = jnp.zeros((ROWS_PER_SUB, D), jnp.float32)
    ones_blk = jnp.ones((BK, D), jnp.float32)

    degp = _deg_kernel(dst, ones_blk, zeros_rows)
    y1 = _tc1(x, W_shared, degp)
    s1p = _segsum_kernel(y1, src, dst, zeros_rows)
    w_cat = jnp.concatenate([W_mu, W_logvar], axis=1)
    b_cat = jnp.concatenate([b_mu, b_logvar])[None, :]
    y2 = _tc2(s1p, y1, degp, w_cat, b_shared[None, :])
    s2p = _segsum_kernel(y2, src, dst, zeros_rows)
    out = _tc3(s2p, y2, degp, b_cat)
    return out[:, :64], out[:, 64:]
